# baseline (speedup 1.0000x reference)
import functools

import jax
import jax.numpy as jnp
from jax import lax
from jax.experimental import pallas as pl
from jax.experimental.pallas import tpu as pltpu
from jax.experimental.pallas import tpu_sc as plsc

N_NODES = 10000
D = 256
H2 = 256
E = 160000
S = 4
L = 128
SP = 8
W = L * SP
NPOS = N_NODES + 16
EP = 160256
E_PER = EP // 32
CH = E_PER // 16
BLK = 1024
CLEN = 5 * BLK
PCH = CLEN // 16
NT = 32
T_OWN = 33
DUMP = 1024
GR = 128
PG = 8
EB = 64
FLUSH_AT = 4096
PD_CAP = 5248
OUT_ROWS = W + 16


def _filter_body(pos_hbm, src_hbm, dst_hbm, ck_hbm, cs_hbm, cnt_hbm,
                 pos_v, src_v, dst_v, ck_v, cs_v, cnt_v):
    c = lax.axis_index("c")
    s = lax.axis_index("s")
    wid = s * 2 + c

    pltpu.sync_copy(pos_hbm, pos_v)
    pltpu.sync_copy(src_hbm.at[pl.ds(wid * E_PER, E_PER)], src_v)
    pltpu.sync_copy(dst_hbm.at[pl.ds(wid * E_PER, E_PER)], dst_v)

    def pf(i, carry):
        ck_v[pl.ds(i * 16, 16)] = jnp.full((16,), DUMP, jnp.int32)
        cs_v[pl.ds(i * 16, 16)] = jnp.zeros((16,), jnp.int32)
        return carry
    lax.fori_loop(0, PCH, pf, 0)

    def fa(i, cur):
        dv = dst_v[pl.ds(i * 16, 16)]
        kv = plsc.load_gather(pos_v, [dv])
        m = kv >= 0
        sv = src_v[pl.ds(i * 16, 16)]
        plsc.store_compressed(ck_v.at[pl.ds(cur, 16)], kv, mask=m)
        plsc.store_compressed(cs_v.at[pl.ds(cur, 16)], sv, mask=m)
        return cur + jnp.sum(jnp.where(m, 1, 0).astype(jnp.int32))
    count = lax.fori_loop(0, CH, fa, jnp.int32(0))

    cnt_v[...] = jnp.full((16,), count, jnp.int32)
    pltpu.sync_copy(ck_v, ck_hbm.at[wid])
    pltpu.sync_copy(cs_v, cs_hbm.at[wid])
    pltpu.sync_copy(cnt_v, cnt_hbm.at[pl.ds(wid * 16, 16)])


def _accum_body(pos_hbm, word_hbm, x_hbm, ck_hbm, cs_hbm, cnt_hbm, parts_hbm,
                pos_v, word_v, cnt_v, kst_v, sst_v, kxt_v, sxt_v, pk_v, ps_v,
                idx_v, rows_v, acc_v, stage_v, jidx_v, wj_v, wk_v, sem):
    c = lax.axis_index("c")
    s = lax.axis_index("s")
    tid = s * 2 + c
    base = tid * T_OWN
    lane = lax.iota(jnp.int32, 16)

    pltpu.sync_copy(pos_hbm, pos_v)
    pltpu.sync_copy(word_hbm, word_v)
    pltpu.sync_copy(cnt_hbm, cnt_v)

    def zb(i, carry):
        acc_v[pl.ds(i * 16, 16)] = jnp.zeros((16,), jnp.float32)
        return carry
    lax.fori_loop(0, (T_OWN + 1) * 16, zb, 0)

    def row_add(r, g):
        half = r // 16
        kchunk = pk_v[pl.ds(g * GR + half * 16, 16)]
        kr = jnp.sum(jnp.where(lane == (r % 16), kchunk, 0))
        local = kr - base
        for j in range(D // 16):
            chunk = rows_v[r, pl.ds(j * 16, 16)]
            plsc.addupdate(acc_v.at[pl.ds(local * D + j * 16, 16)], chunk)
        return g

    def flush_chunk(g, carry):
        for hh in range(GR // 16):
            idx_v[pl.ds(hh * 16, 16)] = ps_v[pl.ds(g * GR + hh * 16, 16)]
        pltpu.async_copy(x_hbm.at[idx_v], rows_v, sem).wait()
        lax.fori_loop(0, GR, row_add, g)
        return carry

    def flush_pending(pcur):
        seal = jnp.full((16,), base + T_OWN, jnp.int32)
        zero = jnp.zeros((16,), jnp.int32)

        def sealw(j, carry):
            pk_v[pl.ds(pcur + j * 16, 16)] = seal
            ps_v[pl.ds(pcur + j * 16, 16)] = zero
            return carry
        lax.fori_loop(0, GR // 16, sealw, 0)
        ng = (pcur + (GR - 1)) // GR
        lax.fori_loop(0, ng, flush_chunk, 0)
        return jnp.int32(0)

    def per_group(gi, pcur0):
        pltpu.sync_copy(ck_hbm.at[pl.ds(gi * PG, PG), pl.ds(0, BLK)], kst_v)
        pltpu.sync_copy(cs_hbm.at[pl.ds(gi * PG, PG), pl.ds(0, BLK)], sst_v)

        def per_p(pi, pcur1):
            p = gi * PG + pi
            cp = jnp.max(cnt_v[pl.ds(p * 16, 16)])
            e0 = jnp.minimum(jnp.int32(BLK), cp)
            nch0 = (e0 + 15) // 16

            def filt(i, cur):
                kv = kst_v[pi, pl.ds(i * 16, 16)]
                m = (kv >= base) & (kv < base + T_OWN)
                sv = sst_v[pi, pl.ds(i * 16, 16)]
                plsc.store_compressed(pk_v.at[pl.ds(cur, 16)], kv, mask=m)
                plsc.store_compressed(ps_v.at[pl.ds(cur, 16)], sv, mask=m)
                return cur + jnp.sum(jnp.where(m, 1, 0).astype(jnp.int32))
            pcur2 = lax.fori_loop(0, nch0, filt, pcur1)
            pcur2 = lax.cond(pcur2 >= FLUSH_AT, flush_pending,
                             lambda cc: cc, pcur2)

            nblk = (cp + (BLK - 1)) // BLK

            def extra(b, pcur3):
                pltpu.sync_copy(ck_hbm.at[p, pl.ds(b * BLK, BLK)], kxt_v)
                pltpu.sync_copy(cs_hbm.at[p, pl.ds(b * BLK, BLK)], sxt_v)
                eb = jnp.minimum(jnp.int32(BLK), cp - b * BLK)
                nch = (eb + 15) // 16

                def filtx(i, cur):
                    kv = kxt_v[pl.ds(i * 16, 16)]
                    m = (kv >= base) & (kv < base + T_OWN)
                    sv = sxt_v[pl.ds(i * 16, 16)]
                    plsc.store_compressed(pk_v.at[pl.ds(cur, 16)], kv, mask=m)
                    plsc.store_compressed(ps_v.at[pl.ds(cur, 16)], sv, mask=m)
                    return cur + jnp.sum(jnp.where(m, 1, 0).astype(jnp.int32))
                pcur4 = lax.fori_loop(0, nch, filtx, pcur3)
                return lax.cond(pcur4 >= FLUSH_AT, flush_pending,
                                lambda cc: cc, pcur4)
            return lax.fori_loop(1, nblk, extra, pcur2)
        return lax.fori_loop(0, PG, per_p, pcur0)
    pend = lax.fori_loop(0, NT // PG, per_group, jnp.int32(0))
    flush_pending(pend)

    def wscan(q, cur):
        wv = word_v[pl.ds(q * 16, 16)]
        kj = plsc.load_gather(pos_v, [wv])
        m = (kj >= base) & (kj < base + T_OWN)
        jv = lane + q * 16
        plsc.store_compressed(wj_v.at[pl.ds(cur, 16)], jv, mask=m)
        plsc.store_compressed(wk_v.at[pl.ds(cur, 16)], kj, mask=m)
        return cur + jnp.sum(jnp.where(m, 1, 0).astype(jnp.int32))
    ccur = lax.fori_loop(0, W // 16, wscan, jnp.int32(0))

    def sealw2(j, carry):
        wj_v[pl.ds(ccur + j * 16, 16)] = jnp.full((16,), W, jnp.int32) + lane
        wk_v[pl.ds(ccur + j * 16, 16)] = jnp.full((16,), base, jnp.int32)
        return carry
    lax.fori_loop(0, EB // 16, sealw2, 0)

    def stage_row(r, g):
        half = r // 16
        kchunk = wk_v[pl.ds(g * EB + half * 16, 16)]
        kr = jnp.sum(jnp.where(lane == (r % 16), kchunk, 0))
        local = kr - base
        for j in range(D // 16):
            stage_v[r, pl.ds(j * 16, 16)] = \
                acc_v[pl.ds(local * D + j * 16, 16)]
        return g

    def emit_chunk(g, carry):
        lax.fori_loop(0, EB, stage_row, g)
        for hh in range(EB // 16):
            jidx_v[pl.ds(hh * 16, 16)] = wj_v[pl.ds(g * EB + hh * 16, 16)]
        pltpu.async_copy(stage_v, parts_hbm.at[jidx_v], sem).wait()
        return carry
    ne = (ccur + (EB - 1)) // EB
    lax.fori_loop(0, ne, emit_chunk, 0)


def _make_filter():
    return functools.partial(
        pl.kernel,
        mesh=plsc.VectorSubcoreMesh(core_axis_name="c", subcore_axis_name="s"),
        out_type=(
            jax.ShapeDtypeStruct((NT, CLEN), jnp.int32),
            jax.ShapeDtypeStruct((NT, CLEN), jnp.int32),
            jax.ShapeDtypeStruct((NT * 16,), jnp.int32),
        ),
        compiler_params=pltpu.CompilerParams(needs_layout_passes=False),
        scratch_types=[
            pltpu.VMEM((NPOS,), jnp.int32),
            pltpu.VMEM((E_PER,), jnp.int32),
            pltpu.VMEM((E_PER,), jnp.int32),
            pltpu.VMEM((CLEN,), jnp.int32),
            pltpu.VMEM((CLEN,), jnp.int32),
            pltpu.VMEM((16,), jnp.int32),
        ],
    )(_filter_body)


def _make_accum():
    return functools.partial(
        pl.kernel,
        mesh=plsc.VectorSubcoreMesh(core_axis_name="c", subcore_axis_name="s"),
        out_type=jax.ShapeDtypeStruct((OUT_ROWS, D), jnp.float32),
        compiler_params=pltpu.CompilerParams(needs_layout_passes=False),
        scratch_types=[
            pltpu.VMEM((NPOS,), jnp.int32),
            pltpu.VMEM((W,), jnp.int32),
            pltpu.VMEM((NT * 16,), jnp.int32),
            pltpu.VMEM((PG, BLK), jnp.int32),
            pltpu.VMEM((PG, BLK), jnp.int32),
            pltpu.VMEM((BLK,), jnp.int32),
            pltpu.VMEM((BLK,), jnp.int32),
            pltpu.VMEM((PD_CAP,), jnp.int32),
            pltpu.VMEM((PD_CAP,), jnp.int32),
            pltpu.VMEM((GR,), jnp.int32),
            pltpu.VMEM((GR, D), jnp.float32),
            pltpu.VMEM(((T_OWN + 1) * D,), jnp.float32),
            pltpu.VMEM((EB, D), jnp.float32),
            pltpu.VMEM((EB,), jnp.int32),
            pltpu.VMEM((W + 2 * EB,), jnp.int32),
            pltpu.VMEM((W + 2 * EB,), jnp.int32),
            pltpu.SemaphoreType.DMA,
        ],
    )(_accum_body)


def _tc_body(parts_ref, w1_ref, b1_ref, wih_ref, whh_ref, bih_ref, bhh_ref,
             ws_ref, bs_ref, out_ref, xp_ref):
    we = parts_ref[pl.ds(0, W), :]
    hw = lax.dot_general(we, w1_ref[...], (((1,), (1,)), ((), ())),
                         preferred_element_type=jnp.float32) + b1_ref[...]
    xp_ref[...] = lax.dot_general(hw, wih_ref[...], (((1,), (1,)), ((), ())),
                                  preferred_element_type=jnp.float32) \
        + bih_ref[...] + bhh_ref[...]

    def step(t, hc):
        h, cc = hc
        g = xp_ref[pl.ds(t * SP, SP), :] + lax.dot_general(
            h, whh_ref[...], (((1,), (1,)), ((), ())),
            preferred_element_type=jnp.float32)
        ii = jax.nn.sigmoid(g[:, 0:H2])
        ff = jax.nn.sigmoid(g[:, H2:2 * H2])
        gg = jnp.tanh(g[:, 2 * H2:3 * H2])
        oo = jax.nn.sigmoid(g[:, 3 * H2:4 * H2])
        cn = ff * cc + ii * gg
        hn = oo * jnp.tanh(cn)
        return (hn, cn)

    h0 = jnp.zeros((SP, H2), jnp.float32)
    h, _ = lax.fori_loop(0, L, step, (h0, h0))
    sent = jnp.maximum(h, 0.0)
    out_ref[...] = lax.dot_general(sent, ws_ref[...], (((1,), (1,)), ((), ())),
                                   preferred_element_type=jnp.float32) + bs_ref[...]


def kernel(x, edge_index, word_idx_list, W1, b1, W_ih, W_hh, b_ih, b_hh, Ws, bs):
    wp = jnp.zeros((L, SP), jnp.int32).at[:, :S].set(
        word_idx_list.astype(jnp.int32).T)
    word_flat = wp.reshape(-1)
    pos = jnp.full((NPOS,), -1, jnp.int32).at[word_flat].set(
        jnp.arange(W, dtype=jnp.int32))
    src_p = jnp.concatenate(
        [edge_index[0].astype(jnp.int32), jnp.zeros((EP - E,), jnp.int32)])
    dst_p = jnp.concatenate(
        [edge_index[1].astype(jnp.int32),
         jnp.full((EP - E,), N_NODES, jnp.int32)])

    ck, cs, cnt = _make_filter()(pos, src_p, dst_p)
    parts = _make_accum()(pos, word_flat, x, ck, cs, cnt)

    scores8 = pl.pallas_call(
        _tc_body,
        out_shape=jax.ShapeDtypeStruct((SP, 2), jnp.float32),
        scratch_shapes=[pltpu.VMEM((W, 4 * H2), jnp.float32)],
    )(parts, W1, b1.reshape(1, -1), W_ih, W_hh, b_ih.reshape(1, -1),
      b_hh.reshape(1, -1), Ws, bs.reshape(1, -1))
    return scores8[:S]


# speedup vs baseline: 1.1491x; 1.1491x over previous
import functools

import jax
import jax.numpy as jnp
from jax import lax
from jax.experimental import pallas as pl
from jax.experimental.pallas import tpu as pltpu
from jax.experimental.pallas import tpu_sc as plsc

N_NODES = 10000
D = 256
H2 = 256
E = 160000
S = 4
L = 128
SP = 8
W = L * SP
NPOS = N_NODES + 16
EP = 160256
E_PER = EP // 32
CH = E_PER // 16
BLK = 1024
CLEN = 5 * BLK
PCH = CLEN // 16
NT = 32
T_OWN = 33
DUMP = 1024
GR = 64
PG = 8
EB = 64
FLUSH_AT = 4096
PD_CAP = 5248
OUT_ROWS = W + 16


def _filter_body(pos_hbm, src_hbm, dst_hbm, ck_hbm, cs_hbm, cnt_hbm,
                 pos_v, src_v, dst_v, ck_v, cs_v, cnt_v):
    c = lax.axis_index("c")
    s = lax.axis_index("s")
    wid = s * 2 + c

    pltpu.sync_copy(pos_hbm, pos_v)
    pltpu.sync_copy(src_hbm.at[pl.ds(wid * E_PER, E_PER)], src_v)
    pltpu.sync_copy(dst_hbm.at[pl.ds(wid * E_PER, E_PER)], dst_v)

    def pf(i, carry):
        ck_v[pl.ds(i * 16, 16)] = jnp.full((16,), DUMP, jnp.int32)
        cs_v[pl.ds(i * 16, 16)] = jnp.zeros((16,), jnp.int32)
        return carry
    lax.fori_loop(0, PCH, pf, 0)

    def fa(i, cur):
        dv = dst_v[pl.ds(i * 16, 16)]
        kv = plsc.load_gather(pos_v, [dv])
        m = kv >= 0
        sv = src_v[pl.ds(i * 16, 16)]
        plsc.store_compressed(ck_v.at[pl.ds(cur, 16)], kv, mask=m)
        plsc.store_compressed(cs_v.at[pl.ds(cur, 16)], sv, mask=m)
        return cur + jnp.sum(jnp.where(m, 1, 0).astype(jnp.int32))
    count = lax.fori_loop(0, CH, fa, jnp.int32(0))

    cnt_v[...] = jnp.full((16,), count, jnp.int32)
    pltpu.sync_copy(ck_v, ck_hbm.at[wid])
    pltpu.sync_copy(cs_v, cs_hbm.at[wid])
    pltpu.sync_copy(cnt_v, cnt_hbm.at[pl.ds(wid * 16, 16)])


def _accum_body(pos_hbm, word_hbm, x_hbm, ck_hbm, cs_hbm, cnt_hbm, parts_hbm,
                pos_v, word_v, cnt_v, kst_v, sst_v, kxt_v, sxt_v, pk_v, ps_v,
                idx_a, idx_b, rows_a, rows_b, acc_v, stage_v, jidx_v,
                wj_v, wk_v, sem, sem_a, sem_b):
    c = lax.axis_index("c")
    s = lax.axis_index("s")
    tid = s * 2 + c
    base = tid * T_OWN
    lane = lax.iota(jnp.int32, 16)

    pltpu.sync_copy(pos_hbm, pos_v)
    pltpu.sync_copy(word_hbm, word_v)
    pltpu.sync_copy(cnt_hbm, cnt_v)

    def zb(i, carry):
        acc_v[pl.ds(i * 16, 16)] = jnp.zeros((16,), jnp.float32)
        return carry
    lax.fori_loop(0, (T_OWN + 1) * 16, zb, 0)

    def _mk_row_add(rows_ref):
        def row_add(r, g):
            half = r // 16
            kchunk = pk_v[pl.ds(g * GR + half * 16, 16)]
            kr = jnp.sum(jnp.where(lane == (r % 16), kchunk, 0))
            local = kr - base
            for j in range(D // 16):
                chunk = rows_ref[r, pl.ds(j * 16, 16)]
                plsc.addupdate(acc_v.at[pl.ds(local * D + j * 16, 16)], chunk)
            return g
        return row_add
    row_add_a = _mk_row_add(rows_a)
    row_add_b = _mk_row_add(rows_b)

    def _fire(idx_ref, rows_ref, sm):
        def fire(g):
            for hh in range(GR // 16):
                idx_ref[pl.ds(hh * 16, 16)] = ps_v[pl.ds(g * GR + hh * 16, 16)]
            pltpu.async_copy(x_hbm.at[idx_ref], rows_ref, sm)
            return jnp.int32(0)
        return fire
    fire_a = _fire(idx_a, rows_a, sem_a)
    fire_b = _fire(idx_b, rows_b, sem_b)

    def _drain(rows_ref, sm):
        def drain():
            pltpu.make_async_copy(
                x_hbm.at[pl.ds(0, GR)], rows_ref, sm).wait()
        return drain
    drain_a = _drain(rows_a, sem_a)
    drain_b = _drain(rows_b, sem_b)

    def flush_pending(pcur):
        seal = jnp.full((16,), base + T_OWN, jnp.int32)
        zero = jnp.zeros((16,), jnp.int32)

        def sealw(j, carry):
            pk_v[pl.ds(pcur + j * 16, 16)] = seal
            ps_v[pl.ds(pcur + j * 16, 16)] = zero
            return carry
        lax.fori_loop(0, GR // 16, sealw, 0)
        ng = (pcur + (GR - 1)) // GR

        lax.cond(ng > 0, fire_a, lambda g: jnp.int32(0), jnp.int32(0))

        def pair(u, carry):
            g1 = 2 * u + 1
            g2 = 2 * u + 2
            lax.cond(g1 < ng, fire_b, lambda g: jnp.int32(0), g1)
            drain_a()
            lax.fori_loop(0, GR, row_add_a, 2 * u)

            def do_b(_):
                lax.cond(g2 < ng, fire_a, lambda g: jnp.int32(0), g2)
                drain_b()
                lax.fori_loop(0, GR, row_add_b, g1)
                return jnp.int32(0)
            lax.cond(g1 < ng, do_b, lambda _: jnp.int32(0), 0)
            return carry
        lax.fori_loop(0, (ng + 1) // 2, pair, 0)
        return jnp.int32(0)

    def per_group(gi, pcur0):
        pltpu.sync_copy(ck_hbm.at[pl.ds(gi * PG, PG), pl.ds(0, BLK)], kst_v)
        pltpu.sync_copy(cs_hbm.at[pl.ds(gi * PG, PG), pl.ds(0, BLK)], sst_v)

        def per_p(pi, pcur1):
            p = gi * PG + pi
            cp = jnp.max(cnt_v[pl.ds(p * 16, 16)])
            e0 = jnp.minimum(jnp.int32(BLK), cp)
            nch0 = (e0 + 15) // 16

            def filt(i, cur):
                kv = kst_v[pi, pl.ds(i * 16, 16)]
                m = (kv >= base) & (kv < base + T_OWN)
                sv = sst_v[pi, pl.ds(i * 16, 16)]
                plsc.store_compressed(pk_v.at[pl.ds(cur, 16)], kv, mask=m)
                plsc.store_compressed(ps_v.at[pl.ds(cur, 16)], sv, mask=m)
                return cur + jnp.sum(jnp.where(m, 1, 0).astype(jnp.int32))
            pcur2 = lax.fori_loop(0, nch0, filt, pcur1)
            pcur2 = lax.cond(pcur2 >= FLUSH_AT, flush_pending,
                             lambda cc: cc, pcur2)

            nblk = (cp + (BLK - 1)) // BLK

            def extra(b, pcur3):
                pltpu.sync_copy(ck_hbm.at[p, pl.ds(b * BLK, BLK)], kxt_v)
                pltpu.sync_copy(cs_hbm.at[p, pl.ds(b * BLK, BLK)], sxt_v)
                eb = jnp.minimum(jnp.int32(BLK), cp - b * BLK)
                nch = (eb + 15) // 16

                def filtx(i, cur):
                    kv = kxt_v[pl.ds(i * 16, 16)]
                    m = (kv >= base) & (kv < base + T_OWN)
                    sv = sxt_v[pl.ds(i * 16, 16)]
                    plsc.store_compressed(pk_v.at[pl.ds(cur, 16)], kv, mask=m)
                    plsc.store_compressed(ps_v.at[pl.ds(cur, 16)], sv, mask=m)
                    return cur + jnp.sum(jnp.where(m, 1, 0).astype(jnp.int32))
                pcur4 = lax.fori_loop(0, nch, filtx, pcur3)
                return lax.cond(pcur4 >= FLUSH_AT, flush_pending,
                                lambda cc: cc, pcur4)
            return lax.fori_loop(1, nblk, extra, pcur2)
        return lax.fori_loop(0, PG, per_p, pcur0)
    pend = lax.fori_loop(0, NT // PG, per_group, jnp.int32(0))
    flush_pending(pend)

    def wscan(q, cur):
        wv = word_v[pl.ds(q * 16, 16)]
        kj = plsc.load_gather(pos_v, [wv])
        m = (kj >= base) & (kj < base + T_OWN)
        jv = lane + q * 16
        plsc.store_compressed(wj_v.at[pl.ds(cur, 16)], jv, mask=m)
        plsc.store_compressed(wk_v.at[pl.ds(cur, 16)], kj, mask=m)
        return cur + jnp.sum(jnp.where(m, 1, 0).astype(jnp.int32))
    ccur = lax.fori_loop(0, W // 16, wscan, jnp.int32(0))

    def sealw2(j, carry):
        wj_v[pl.ds(ccur + j * 16, 16)] = jnp.full((16,), W, jnp.int32) + lane
        wk_v[pl.ds(ccur + j * 16, 16)] = jnp.full((16,), base, jnp.int32)
        return carry
    lax.fori_loop(0, EB // 16, sealw2, 0)

    def stage_row(r, g):
        half = r // 16
        kchunk = wk_v[pl.ds(g * EB + half * 16, 16)]
        kr = jnp.sum(jnp.where(lane == (r % 16), kchunk, 0))
        local = kr - base
        for j in range(D // 16):
            stage_v[r, pl.ds(j * 16, 16)] = \
                acc_v[pl.ds(local * D + j * 16, 16)]
        return g

    def emit_chunk(g, carry):
        lax.fori_loop(0, EB, stage_row, g)
        for hh in range(EB // 16):
            jidx_v[pl.ds(hh * 16, 16)] = wj_v[pl.ds(g * EB + hh * 16, 16)]
        pltpu.async_copy(stage_v, parts_hbm.at[jidx_v], sem).wait()
        return carry
    ne = (ccur + (EB - 1)) // EB
    lax.fori_loop(0, ne, emit_chunk, 0)


def _make_filter():
    return functools.partial(
        pl.kernel,
        mesh=plsc.VectorSubcoreMesh(core_axis_name="c", subcore_axis_name="s"),
        out_type=(
            jax.ShapeDtypeStruct((NT, CLEN), jnp.int32),
            jax.ShapeDtypeStruct((NT, CLEN), jnp.int32),
            jax.ShapeDtypeStruct((NT * 16,), jnp.int32),
        ),
        compiler_params=pltpu.CompilerParams(needs_layout_passes=False),
        scratch_types=[
            pltpu.VMEM((NPOS,), jnp.int32),
            pltpu.VMEM((E_PER,), jnp.int32),
            pltpu.VMEM((E_PER,), jnp.int32),
            pltpu.VMEM((CLEN,), jnp.int32),
            pltpu.VMEM((CLEN,), jnp.int32),
            pltpu.VMEM((16,), jnp.int32),
        ],
    )(_filter_body)


def _make_accum():
    return functools.partial(
        pl.kernel,
        mesh=plsc.VectorSubcoreMesh(core_axis_name="c", subcore_axis_name="s"),
        out_type=jax.ShapeDtypeStruct((OUT_ROWS, D), jnp.float32),
        compiler_params=pltpu.CompilerParams(needs_layout_passes=False),
        scratch_types=[
            pltpu.VMEM((NPOS,), jnp.int32),
            pltpu.VMEM((W,), jnp.int32),
            pltpu.VMEM((NT * 16,), jnp.int32),
            pltpu.VMEM((PG, BLK), jnp.int32),
            pltpu.VMEM((PG, BLK), jnp.int32),
            pltpu.VMEM((BLK,), jnp.int32),
            pltpu.VMEM((BLK,), jnp.int32),
            pltpu.VMEM((PD_CAP,), jnp.int32),
            pltpu.VMEM((PD_CAP,), jnp.int32),
            pltpu.VMEM((GR,), jnp.int32),
            pltpu.VMEM((GR,), jnp.int32),
            pltpu.VMEM((GR, D), jnp.float32),
            pltpu.VMEM((GR, D), jnp.float32),
            pltpu.VMEM(((T_OWN + 1) * D,), jnp.float32),
            pltpu.VMEM((EB, D), jnp.float32),
            pltpu.VMEM((EB,), jnp.int32),
            pltpu.VMEM((W + 2 * EB,), jnp.int32),
            pltpu.VMEM((W + 2 * EB,), jnp.int32),
            pltpu.SemaphoreType.DMA,
            pltpu.SemaphoreType.DMA,
            pltpu.SemaphoreType.DMA,
        ],
    )(_accum_body)


def _tc_body(parts_ref, w1_ref, b1_ref, wih_ref, whh_ref, bih_ref, bhh_ref,
             ws_ref, bs_ref, out_ref, xp_ref):
    we = parts_ref[pl.ds(0, W), :]
    hw = lax.dot_general(we, w1_ref[...], (((1,), (1,)), ((), ())),
                         preferred_element_type=jnp.float32) + b1_ref[...]
    xp_ref[...] = lax.dot_general(hw, wih_ref[...], (((1,), (1,)), ((), ())),
                                  preferred_element_type=jnp.float32) \
        + bih_ref[...] + bhh_ref[...]

    def step(t, hc):
        h, cc = hc
        g = xp_ref[pl.ds(t * SP, SP), :] + lax.dot_general(
            h, whh_ref[...], (((1,), (1,)), ((), ())),
            preferred_element_type=jnp.float32)
        ii = jax.nn.sigmoid(g[:, 0:H2])
        ff = jax.nn.sigmoid(g[:, H2:2 * H2])
        gg = jnp.tanh(g[:, 2 * H2:3 * H2])
        oo = jax.nn.sigmoid(g[:, 3 * H2:4 * H2])
        cn = ff * cc + ii * gg
        hn = oo * jnp.tanh(cn)
        return (hn, cn)

    h0 = jnp.zeros((SP, H2), jnp.float32)
    h, _ = lax.fori_loop(0, L, step, (h0, h0))
    sent = jnp.maximum(h, 0.0)
    out_ref[...] = lax.dot_general(sent, ws_ref[...], (((1,), (1,)), ((), ())),
                                   preferred_element_type=jnp.float32) + bs_ref[...]


def kernel(x, edge_index, word_idx_list, W1, b1, W_ih, W_hh, b_ih, b_hh, Ws, bs):
    wp = jnp.zeros((L, SP), jnp.int32).at[:, :S].set(
        word_idx_list.astype(jnp.int32).T)
    word_flat = wp.reshape(-1)
    pos = jnp.full((NPOS,), -1, jnp.int32).at[word_flat].set(
        jnp.arange(W, dtype=jnp.int32))
    src_p = jnp.concatenate(
        [edge_index[0].astype(jnp.int32), jnp.zeros((EP - E,), jnp.int32)])
    dst_p = jnp.concatenate(
        [edge_index[1].astype(jnp.int32),
         jnp.full((EP - E,), N_NODES, jnp.int32)])

    ck, cs, cnt = _make_filter()(pos, src_p, dst_p)
    parts = _make_accum()(pos, word_flat, x, ck, cs, cnt)

    scores8 = pl.pallas_call(
        _tc_body,
        out_shape=jax.ShapeDtypeStruct((SP, 2), jnp.float32),
        scratch_shapes=[pltpu.VMEM((W, 4 * H2), jnp.float32)],
    )(parts, W1, b1.reshape(1, -1), W_ih, W_hh, b_ih.reshape(1, -1),
      b_hh.reshape(1, -1), Ws, bs.reshape(1, -1))
    return scores8[:S]


# speedup vs baseline: 1.1551x; 1.0052x over previous
import functools

import jax
import jax.numpy as jnp
from jax import lax
from jax.experimental import pallas as pl
from jax.experimental.pallas import tpu as pltpu
from jax.experimental.pallas import tpu_sc as plsc

N_NODES = 10000
D = 256
H2 = 256
E = 160000
S = 4
L = 128
SP = 8
W = L * SP
NPOS = N_NODES + 16
EP = 160256
E_PER = EP // 32
CH = E_PER // 16
BLK = 1024
CLEN = 5 * BLK
PCH = CLEN // 16
NT = 32
T_OWN = 33
DUMP = 1024
GR = 64
PG = 16
EB = 64
FLUSH_AT = 4096
PD_CAP = 5248
OUT_ROWS = W + 16


def _filter_body(pos_hbm, src_hbm, dst_hbm, ck_hbm, cs_hbm, cnt_hbm,
                 pos_v, src_v, dst_v, ck_v, cs_v, cnt_v):
    c = lax.axis_index("c")
    s = lax.axis_index("s")
    wid = s * 2 + c

    pltpu.sync_copy(pos_hbm, pos_v)
    pltpu.sync_copy(src_hbm.at[pl.ds(wid * E_PER, E_PER)], src_v)
    pltpu.sync_copy(dst_hbm.at[pl.ds(wid * E_PER, E_PER)], dst_v)

    def pf(i, carry):
        ck_v[pl.ds(i * 16, 16)] = jnp.full((16,), DUMP, jnp.int32)
        cs_v[pl.ds(i * 16, 16)] = jnp.zeros((16,), jnp.int32)
        return carry
    lax.fori_loop(0, PCH, pf, 0)

    def fa(i, cur):
        dv = dst_v[pl.ds(i * 16, 16)]
        kv = plsc.load_gather(pos_v, [dv])
        m = kv >= 0
        sv = src_v[pl.ds(i * 16, 16)]
        plsc.store_compressed(ck_v.at[pl.ds(cur, 16)], kv, mask=m)
        plsc.store_compressed(cs_v.at[pl.ds(cur, 16)], sv, mask=m)
        return cur + jnp.sum(jnp.where(m, 1, 0).astype(jnp.int32))
    count = lax.fori_loop(0, CH, fa, jnp.int32(0))

    cnt_v[...] = jnp.full((16,), count, jnp.int32)
    pltpu.sync_copy(ck_v, ck_hbm.at[wid])
    pltpu.sync_copy(cs_v, cs_hbm.at[wid])
    pltpu.sync_copy(cnt_v, cnt_hbm.at[pl.ds(wid * 16, 16)])


def _accum_body(pos_hbm, word_hbm, x_hbm, ck_hbm, cs_hbm, cnt_hbm, parts_hbm,
                pos_v, word_v, cnt_v, kst_v, sst_v, kxt_v, sxt_v, pk_v, ps_v,
                idx_a, idx_b, rows_a, rows_b, acc_v, stage_v, jidx_v,
                wj_v, wk_v, sem, sem_a, sem_b):
    c = lax.axis_index("c")
    s = lax.axis_index("s")
    tid = s * 2 + c
    base = tid * T_OWN
    lane = lax.iota(jnp.int32, 16)

    pltpu.sync_copy(pos_hbm, pos_v)
    pltpu.sync_copy(word_hbm, word_v)
    pltpu.sync_copy(cnt_hbm, cnt_v)

    def zb(i, carry):
        acc_v[pl.ds(i * 16, 16)] = jnp.zeros((16,), jnp.float32)
        return carry
    lax.fori_loop(0, (T_OWN + 1) * 16, zb, 0)

    def _mk_row_add(rows_ref):
        def row_add(r, g):
            half = r // 16
            kchunk = pk_v[pl.ds(g * GR + half * 16, 16)]
            kr = jnp.sum(jnp.where(lane == (r % 16), kchunk, 0))
            local = kr - base
            for j in range(D // 16):
                chunk = rows_ref[r, pl.ds(j * 16, 16)]
                plsc.addupdate(acc_v.at[pl.ds(local * D + j * 16, 16)], chunk)
            return g
        return row_add
    row_add_a = _mk_row_add(rows_a)
    row_add_b = _mk_row_add(rows_b)

    def _fire(idx_ref, rows_ref, sm):
        def fire(g):
            for hh in range(GR // 16):
                idx_ref[pl.ds(hh * 16, 16)] = ps_v[pl.ds(g * GR + hh * 16, 16)]
            pltpu.async_copy(x_hbm.at[idx_ref], rows_ref, sm)
            return jnp.int32(0)
        return fire
    fire_a = _fire(idx_a, rows_a, sem_a)
    fire_b = _fire(idx_b, rows_b, sem_b)

    def _drain(rows_ref, sm):
        def drain():
            pltpu.make_async_copy(
                x_hbm.at[pl.ds(0, GR)], rows_ref, sm).wait()
        return drain
    drain_a = _drain(rows_a, sem_a)
    drain_b = _drain(rows_b, sem_b)

    def flush_pending(pcur):
        seal = jnp.full((16,), base + T_OWN, jnp.int32)
        zero = jnp.zeros((16,), jnp.int32)

        def sealw(j, carry):
            pk_v[pl.ds(pcur + j * 16, 16)] = seal
            ps_v[pl.ds(pcur + j * 16, 16)] = zero
            return carry
        lax.fori_loop(0, GR // 16, sealw, 0)
        ng = (pcur + (GR - 1)) // GR

        lax.cond(ng > 0, fire_a, lambda g: jnp.int32(0), jnp.int32(0))

        def pair(u, carry):
            g1 = 2 * u + 1
            g2 = 2 * u + 2
            lax.cond(g1 < ng, fire_b, lambda g: jnp.int32(0), g1)
            drain_a()
            lax.fori_loop(0, GR, row_add_a, 2 * u)

            def do_b(_):
                lax.cond(g2 < ng, fire_a, lambda g: jnp.int32(0), g2)
                drain_b()
                lax.fori_loop(0, GR, row_add_b, g1)
                return jnp.int32(0)
            lax.cond(g1 < ng, do_b, lambda _: jnp.int32(0), 0)
            return carry
        lax.fori_loop(0, (ng + 1) // 2, pair, 0)
        return jnp.int32(0)

    def per_group(gi, pcur0):
        pltpu.sync_copy(ck_hbm.at[pl.ds(gi * PG, PG), pl.ds(0, BLK)], kst_v)
        pltpu.sync_copy(cs_hbm.at[pl.ds(gi * PG, PG), pl.ds(0, BLK)], sst_v)

        def per_p(pi, pcur1):
            p = gi * PG + pi
            cp = jnp.max(cnt_v[pl.ds(p * 16, 16)])
            e0 = jnp.minimum(jnp.int32(BLK), cp)
            nch0 = (e0 + 15) // 16

            def filt(i, cur):
                kv = kst_v[pi, pl.ds(i * 16, 16)]
                m = (kv >= base) & (kv < base + T_OWN)
                sv = sst_v[pi, pl.ds(i * 16, 16)]
                plsc.store_compressed(pk_v.at[pl.ds(cur, 16)], kv, mask=m)
                plsc.store_compressed(ps_v.at[pl.ds(cur, 16)], sv, mask=m)
                return cur + jnp.sum(jnp.where(m, 1, 0).astype(jnp.int32))
            pcur2 = lax.fori_loop(0, nch0, filt, pcur1)
            pcur2 = lax.cond(pcur2 >= FLUSH_AT, flush_pending,
                             lambda cc: cc, pcur2)

            nblk = (cp + (BLK - 1)) // BLK

            def extra(b, pcur3):
                pltpu.sync_copy(ck_hbm.at[p, pl.ds(b * BLK, BLK)], kxt_v)
                pltpu.sync_copy(cs_hbm.at[p, pl.ds(b * BLK, BLK)], sxt_v)
                eb = jnp.minimum(jnp.int32(BLK), cp - b * BLK)
                nch = (eb + 15) // 16

                def filtx(i, cur):
                    kv = kxt_v[pl.ds(i * 16, 16)]
                    m = (kv >= base) & (kv < base + T_OWN)
                    sv = sxt_v[pl.ds(i * 16, 16)]
                    plsc.store_compressed(pk_v.at[pl.ds(cur, 16)], kv, mask=m)
                    plsc.store_compressed(ps_v.at[pl.ds(cur, 16)], sv, mask=m)
                    return cur + jnp.sum(jnp.where(m, 1, 0).astype(jnp.int32))
                pcur4 = lax.fori_loop(0, nch, filtx, pcur3)
                return lax.cond(pcur4 >= FLUSH_AT, flush_pending,
                                lambda cc: cc, pcur4)
            return lax.fori_loop(1, nblk, extra, pcur2)
        return lax.fori_loop(0, PG, per_p, pcur0)
    pend = lax.fori_loop(0, NT // PG, per_group, jnp.int32(0))
    flush_pending(pend)

    def wscan(q, cur):
        wv = word_v[pl.ds(q * 16, 16)]
        kj = plsc.load_gather(pos_v, [wv])
        m = (kj >= base) & (kj < base + T_OWN)
        jv = lane + q * 16
        plsc.store_compressed(wj_v.at[pl.ds(cur, 16)], jv, mask=m)
        plsc.store_compressed(wk_v.at[pl.ds(cur, 16)], kj, mask=m)
        return cur + jnp.sum(jnp.where(m, 1, 0).astype(jnp.int32))
    ccur = lax.fori_loop(0, W // 16, wscan, jnp.int32(0))

    def sealw2(j, carry):
        wj_v[pl.ds(ccur + j * 16, 16)] = jnp.full((16,), W, jnp.int32) + lane
        wk_v[pl.ds(ccur + j * 16, 16)] = jnp.full((16,), base, jnp.int32)
        return carry
    lax.fori_loop(0, EB // 16, sealw2, 0)

    def stage_row(r, g):
        half = r // 16
        kchunk = wk_v[pl.ds(g * EB + half * 16, 16)]
        kr = jnp.sum(jnp.where(lane == (r % 16), kchunk, 0))
        local = kr - base
        for j in range(D // 16):
            stage_v[r, pl.ds(j * 16, 16)] = \
                acc_v[pl.ds(local * D + j * 16, 16)]
        return g

    def emit_chunk(g, carry):
        lax.fori_loop(0, EB, stage_row, g)
        for hh in range(EB // 16):
            jidx_v[pl.ds(hh * 16, 16)] = wj_v[pl.ds(g * EB + hh * 16, 16)]
        pltpu.async_copy(stage_v, parts_hbm.at[jidx_v], sem).wait()
        return carry
    ne = (ccur + (EB - 1)) // EB
    lax.fori_loop(0, ne, emit_chunk, 0)


def _make_filter():
    return functools.partial(
        pl.kernel,
        mesh=plsc.VectorSubcoreMesh(core_axis_name="c", subcore_axis_name="s"),
        out_type=(
            jax.ShapeDtypeStruct((NT, CLEN), jnp.int32),
            jax.ShapeDtypeStruct((NT, CLEN), jnp.int32),
            jax.ShapeDtypeStruct((NT * 16,), jnp.int32),
        ),
        compiler_params=pltpu.CompilerParams(needs_layout_passes=False),
        scratch_types=[
            pltpu.VMEM((NPOS,), jnp.int32),
            pltpu.VMEM((E_PER,), jnp.int32),
            pltpu.VMEM((E_PER,), jnp.int32),
            pltpu.VMEM((CLEN,), jnp.int32),
            pltpu.VMEM((CLEN,), jnp.int32),
            pltpu.VMEM((16,), jnp.int32),
        ],
    )(_filter_body)


def _make_accum():
    return functools.partial(
        pl.kernel,
        mesh=plsc.VectorSubcoreMesh(core_axis_name="c", subcore_axis_name="s"),
        out_type=jax.ShapeDtypeStruct((OUT_ROWS, D), jnp.float32),
        compiler_params=pltpu.CompilerParams(needs_layout_passes=False),
        scratch_types=[
            pltpu.VMEM((NPOS,), jnp.int32),
            pltpu.VMEM((W,), jnp.int32),
            pltpu.VMEM((NT * 16,), jnp.int32),
            pltpu.VMEM((PG, BLK), jnp.int32),
            pltpu.VMEM((PG, BLK), jnp.int32),
            pltpu.VMEM((BLK,), jnp.int32),
            pltpu.VMEM((BLK,), jnp.int32),
            pltpu.VMEM((PD_CAP,), jnp.int32),
            pltpu.VMEM((PD_CAP,), jnp.int32),
            pltpu.VMEM((GR,), jnp.int32),
            pltpu.VMEM((GR,), jnp.int32),
            pltpu.VMEM((GR, D), jnp.float32),
            pltpu.VMEM((GR, D), jnp.float32),
            pltpu.VMEM(((T_OWN + 1) * D,), jnp.float32),
            pltpu.VMEM((EB, D), jnp.float32),
            pltpu.VMEM((EB,), jnp.int32),
            pltpu.VMEM((W + 2 * EB,), jnp.int32),
            pltpu.VMEM((W + 2 * EB,), jnp.int32),
            pltpu.SemaphoreType.DMA,
            pltpu.SemaphoreType.DMA,
            pltpu.SemaphoreType.DMA,
        ],
    )(_accum_body)


def _tc_body(parts_ref, w1_ref, b1_ref, wih_ref, whh_ref, bih_ref, bhh_ref,
             ws_ref, bs_ref, out_ref, xp_ref):
    we = parts_ref[pl.ds(0, W), :]
    hw = lax.dot_general(we, w1_ref[...], (((1,), (1,)), ((), ())),
                         preferred_element_type=jnp.float32) + b1_ref[...]
    xp_ref[...] = lax.dot_general(hw, wih_ref[...], (((1,), (1,)), ((), ())),
                                  preferred_element_type=jnp.float32) \
        + bih_ref[...] + bhh_ref[...]

    def step(t, hc):
        h, cc = hc
        g = xp_ref[pl.ds(t * SP, SP), :] + lax.dot_general(
            h, whh_ref[...], (((1,), (1,)), ((), ())),
            preferred_element_type=jnp.float32)
        ii = jax.nn.sigmoid(g[:, 0:H2])
        ff = jax.nn.sigmoid(g[:, H2:2 * H2])
        gg = jnp.tanh(g[:, 2 * H2:3 * H2])
        oo = jax.nn.sigmoid(g[:, 3 * H2:4 * H2])
        cn = ff * cc + ii * gg
        hn = oo * jnp.tanh(cn)
        return (hn, cn)

    h0 = jnp.zeros((SP, H2), jnp.float32)
    h, _ = lax.fori_loop(0, L, step, (h0, h0))
    sent = jnp.maximum(h, 0.0)
    out_ref[...] = lax.dot_general(sent, ws_ref[...], (((1,), (1,)), ((), ())),
                                   preferred_element_type=jnp.float32) + bs_ref[...]


def kernel(x, edge_index, word_idx_list, W1, b1, W_ih, W_hh, b_ih, b_hh, Ws, bs):
    wp = jnp.zeros((L, SP), jnp.int32).at[:, :S].set(
        word_idx_list.astype(jnp.int32).T)
    word_flat = wp.reshape(-1)
    pos = jnp.full((NPOS,), -1, jnp.int32).at[word_flat].set(
        jnp.arange(W, dtype=jnp.int32))
    src_p = jnp.concatenate(
        [edge_index[0].astype(jnp.int32), jnp.zeros((EP - E,), jnp.int32)])
    dst_p = jnp.concatenate(
        [edge_index[1].astype(jnp.int32),
         jnp.full((EP - E,), N_NODES, jnp.int32)])

    ck, cs, cnt = _make_filter()(pos, src_p, dst_p)
    parts = _make_accum()(pos, word_flat, x, ck, cs, cnt)

    scores8 = pl.pallas_call(
        _tc_body,
        out_shape=jax.ShapeDtypeStruct((SP, 2), jnp.float32),
        scratch_shapes=[pltpu.VMEM((W, 4 * H2), jnp.float32)],
    )(parts, W1, b1.reshape(1, -1), W_ih, W_hh, b_ih.reshape(1, -1),
      b_hh.reshape(1, -1), Ws, bs.reshape(1, -1))
    return scores8[:S]


# speedup vs baseline: 1.1770x; 1.0189x over previous
import functools

import jax
import jax.numpy as jnp
from jax import lax
from jax.experimental import pallas as pl
from jax.experimental.pallas import tpu as pltpu
from jax.experimental.pallas import tpu_sc as plsc

N_NODES = 10000
D = 256
H2 = 256
E = 160000
S = 4
L = 128
SP = 8
W = L * SP
NPOS = N_NODES + 16
EP = 160256
E_PER = EP // 32
CH = E_PER // 16
BLK = 1024
CLEN = 5 * BLK
PCH = CLEN // 16
NT = 32
T_OWN = 33
DUMP = 1024
GR = 64
PG = 16
EB = 64
FLUSH_AT = 4096
PD_CAP = 5248
OUT_ROWS = W + 16


def _filter_body(pos_hbm, src_hbm, dst_hbm, ck_hbm, cs_hbm, cnt_hbm,
                 pos_v, src_v, dst_v, ck_v, cs_v, cnt_v):
    c = lax.axis_index("c")
    s = lax.axis_index("s")
    wid = s * 2 + c

    pltpu.sync_copy(pos_hbm, pos_v)
    pltpu.sync_copy(src_hbm.at[pl.ds(wid * E_PER, E_PER)], src_v)
    pltpu.sync_copy(dst_hbm.at[pl.ds(wid * E_PER, E_PER)], dst_v)

    def pf(i, carry):
        ck_v[pl.ds(i * 16, 16)] = jnp.full((16,), DUMP, jnp.int32)
        cs_v[pl.ds(i * 16, 16)] = jnp.zeros((16,), jnp.int32)
        return carry
    lax.fori_loop(0, PCH, pf, 0)

    def fa(i, cur):
        dv = dst_v[pl.ds(i * 16, 16)]
        kv = plsc.load_gather(pos_v, [dv])
        m = kv >= 0
        sv = src_v[pl.ds(i * 16, 16)]
        plsc.store_compressed(ck_v.at[pl.ds(cur, 16)], kv, mask=m)
        plsc.store_compressed(cs_v.at[pl.ds(cur, 16)], sv, mask=m)
        return cur + jnp.sum(jnp.where(m, 1, 0).astype(jnp.int32))
    count = lax.fori_loop(0, CH, fa, jnp.int32(0))

    cnt_v[...] = jnp.full((16,), count, jnp.int32)
    pltpu.sync_copy(ck_v, ck_hbm.at[wid])
    pltpu.sync_copy(cs_v, cs_hbm.at[wid])
    pltpu.sync_copy(cnt_v, cnt_hbm.at[pl.ds(wid * 16, 16)])


def _accum_body(pos_hbm, word_hbm, x_hbm, ck_hbm, cs_hbm, cnt_hbm, parts_hbm,
                pos_v, word_v, cnt_v, kst_v, sst_v, kxt_v, sxt_v, pk_v, ps_v,
                idx_a, idx_b, rows_a, rows_b, acc_v, stage_v, jidx_v,
                wj_v, wk_v, sem, sem_a, sem_b):
    c = lax.axis_index("c")
    s = lax.axis_index("s")
    tid = s * 2 + c
    base = tid * T_OWN
    lane = lax.iota(jnp.int32, 16)

    pltpu.sync_copy(pos_hbm, pos_v)
    pltpu.sync_copy(word_hbm, word_v)
    pltpu.sync_copy(cnt_hbm, cnt_v)

    def zb(i, carry):
        acc_v[pl.ds(i * 16, 16)] = jnp.zeros((16,), jnp.float32)
        return carry
    lax.fori_loop(0, (T_OWN + 1) * 16, zb, 0)

    def _mk_row_add(rows_ref):
        def row_add(r, g):
            kv = pk_v[pl.ds(g * GR + r, 16)]
            kr = kv[0]
            local = kr - base
            for j in range(D // 16):
                chunk = rows_ref[r, pl.ds(j * 16, 16)]
                plsc.addupdate(acc_v.at[pl.ds(local * D + j * 16, 16)], chunk)
            return g
        return row_add
    row_add_a = _mk_row_add(rows_a)
    row_add_b = _mk_row_add(rows_b)

    def _fire(idx_ref, rows_ref, sm):
        def fire(g):
            for hh in range(GR // 16):
                idx_ref[pl.ds(hh * 16, 16)] = ps_v[pl.ds(g * GR + hh * 16, 16)]
            pltpu.async_copy(x_hbm.at[idx_ref], rows_ref, sm)
            return jnp.int32(0)
        return fire
    fire_a = _fire(idx_a, rows_a, sem_a)
    fire_b = _fire(idx_b, rows_b, sem_b)

    def _drain(rows_ref, sm):
        def drain():
            pltpu.make_async_copy(
                x_hbm.at[pl.ds(0, GR)], rows_ref, sm).wait()
        return drain
    drain_a = _drain(rows_a, sem_a)
    drain_b = _drain(rows_b, sem_b)

    def flush_pending(pcur):
        seal = jnp.full((16,), base + T_OWN, jnp.int32)
        zero = jnp.zeros((16,), jnp.int32)

        def sealw(j, carry):
            pk_v[pl.ds(pcur + j * 16, 16)] = seal
            ps_v[pl.ds(pcur + j * 16, 16)] = zero
            return carry
        lax.fori_loop(0, GR // 16, sealw, 0)
        ng = (pcur + (GR - 1)) // GR

        lax.cond(ng > 0, fire_a, lambda g: jnp.int32(0), jnp.int32(0))

        def pair(u, carry):
            g1 = 2 * u + 1
            g2 = 2 * u + 2
            lax.cond(g1 < ng, fire_b, lambda g: jnp.int32(0), g1)
            drain_a()
            lax.fori_loop(0, GR, row_add_a, 2 * u)

            def do_b(_):
                lax.cond(g2 < ng, fire_a, lambda g: jnp.int32(0), g2)
                drain_b()
                lax.fori_loop(0, GR, row_add_b, g1)
                return jnp.int32(0)
            lax.cond(g1 < ng, do_b, lambda _: jnp.int32(0), 0)
            return carry
        lax.fori_loop(0, (ng + 1) // 2, pair, 0)
        return jnp.int32(0)

    def per_group(gi, pcur0):
        pltpu.sync_copy(ck_hbm.at[pl.ds(gi * PG, PG), pl.ds(0, BLK)], kst_v)
        pltpu.sync_copy(cs_hbm.at[pl.ds(gi * PG, PG), pl.ds(0, BLK)], sst_v)

        def per_p(pi, pcur1):
            p = gi * PG + pi
            cp = jnp.max(cnt_v[pl.ds(p * 16, 16)])
            e0 = jnp.minimum(jnp.int32(BLK), cp)
            nch0 = (e0 + 15) // 16

            def filt(i, cur):
                kv = kst_v[pi, pl.ds(i * 16, 16)]
                m = (kv >= base) & (kv < base + T_OWN)
                sv = sst_v[pi, pl.ds(i * 16, 16)]
                plsc.store_compressed(pk_v.at[pl.ds(cur, 16)], kv, mask=m)
                plsc.store_compressed(ps_v.at[pl.ds(cur, 16)], sv, mask=m)
                return cur + jnp.sum(jnp.where(m, 1, 0).astype(jnp.int32))
            pcur2 = lax.fori_loop(0, nch0, filt, pcur1)
            pcur2 = lax.cond(pcur2 >= FLUSH_AT, flush_pending,
                             lambda cc: cc, pcur2)

            nblk = (cp + (BLK - 1)) // BLK

            def extra(b, pcur3):
                pltpu.sync_copy(ck_hbm.at[p, pl.ds(b * BLK, BLK)], kxt_v)
                pltpu.sync_copy(cs_hbm.at[p, pl.ds(b * BLK, BLK)], sxt_v)
                eb = jnp.minimum(jnp.int32(BLK), cp - b * BLK)
                nch = (eb + 15) // 16

                def filtx(i, cur):
                    kv = kxt_v[pl.ds(i * 16, 16)]
                    m = (kv >= base) & (kv < base + T_OWN)
                    sv = sxt_v[pl.ds(i * 16, 16)]
                    plsc.store_compressed(pk_v.at[pl.ds(cur, 16)], kv, mask=m)
                    plsc.store_compressed(ps_v.at[pl.ds(cur, 16)], sv, mask=m)
                    return cur + jnp.sum(jnp.where(m, 1, 0).astype(jnp.int32))
                pcur4 = lax.fori_loop(0, nch, filtx, pcur3)
                return lax.cond(pcur4 >= FLUSH_AT, flush_pending,
                                lambda cc: cc, pcur4)
            return lax.fori_loop(1, nblk, extra, pcur2)
        return lax.fori_loop(0, PG, per_p, pcur0)
    pend = lax.fori_loop(0, NT // PG, per_group, jnp.int32(0))
    flush_pending(pend)

    def wscan(q, cur):
        wv = word_v[pl.ds(q * 16, 16)]
        kj = plsc.load_gather(pos_v, [wv])
        m = (kj >= base) & (kj < base + T_OWN)
        jv = lane + q * 16
        plsc.store_compressed(wj_v.at[pl.ds(cur, 16)], jv, mask=m)
        plsc.store_compressed(wk_v.at[pl.ds(cur, 16)], kj, mask=m)
        return cur + jnp.sum(jnp.where(m, 1, 0).astype(jnp.int32))
    ccur = lax.fori_loop(0, W // 16, wscan, jnp.int32(0))

    def sealw2(j, carry):
        wj_v[pl.ds(ccur + j * 16, 16)] = jnp.full((16,), W, jnp.int32) + lane
        wk_v[pl.ds(ccur + j * 16, 16)] = jnp.full((16,), base, jnp.int32)
        return carry
    lax.fori_loop(0, EB // 16, sealw2, 0)

    def stage_row(r, g):
        kv = wk_v[pl.ds(g * EB + r, 16)]
        kr = kv[0]
        local = kr - base
        for j in range(D // 16):
            stage_v[r, pl.ds(j * 16, 16)] = \
                acc_v[pl.ds(local * D + j * 16, 16)]
        return g

    def emit_chunk(g, carry):
        lax.fori_loop(0, EB, stage_row, g)
        for hh in range(EB // 16):
            jidx_v[pl.ds(hh * 16, 16)] = wj_v[pl.ds(g * EB + hh * 16, 16)]
        pltpu.async_copy(stage_v, parts_hbm.at[jidx_v], sem).wait()
        return carry
    ne = (ccur + (EB - 1)) // EB
    lax.fori_loop(0, ne, emit_chunk, 0)


def _make_filter():
    return functools.partial(
        pl.kernel,
        mesh=plsc.VectorSubcoreMesh(core_axis_name="c", subcore_axis_name="s"),
        out_type=(
            jax.ShapeDtypeStruct((NT, CLEN), jnp.int32),
            jax.ShapeDtypeStruct((NT, CLEN), jnp.int32),
            jax.ShapeDtypeStruct((NT * 16,), jnp.int32),
        ),
        compiler_params=pltpu.CompilerParams(needs_layout_passes=False),
        scratch_types=[
            pltpu.VMEM((NPOS,), jnp.int32),
            pltpu.VMEM((E_PER,), jnp.int32),
            pltpu.VMEM((E_PER,), jnp.int32),
            pltpu.VMEM((CLEN,), jnp.int32),
            pltpu.VMEM((CLEN,), jnp.int32),
            pltpu.VMEM((16,), jnp.int32),
        ],
    )(_filter_body)


def _make_accum():
    return functools.partial(
        pl.kernel,
        mesh=plsc.VectorSubcoreMesh(core_axis_name="c", subcore_axis_name="s"),
        out_type=jax.ShapeDtypeStruct((OUT_ROWS, D), jnp.float32),
        compiler_params=pltpu.CompilerParams(needs_layout_passes=False),
        scratch_types=[
            pltpu.VMEM((NPOS,), jnp.int32),
            pltpu.VMEM((W,), jnp.int32),
            pltpu.VMEM((NT * 16,), jnp.int32),
            pltpu.VMEM((PG, BLK), jnp.int32),
            pltpu.VMEM((PG, BLK), jnp.int32),
            pltpu.VMEM((BLK,), jnp.int32),
            pltpu.VMEM((BLK,), jnp.int32),
            pltpu.VMEM((PD_CAP,), jnp.int32),
            pltpu.VMEM((PD_CAP,), jnp.int32),
            pltpu.VMEM((GR,), jnp.int32),
            pltpu.VMEM((GR,), jnp.int32),
            pltpu.VMEM((GR, D), jnp.float32),
            pltpu.VMEM((GR, D), jnp.float32),
            pltpu.VMEM(((T_OWN + 1) * D,), jnp.float32),
            pltpu.VMEM((EB, D), jnp.float32),
            pltpu.VMEM((EB,), jnp.int32),
            pltpu.VMEM((W + 2 * EB,), jnp.int32),
            pltpu.VMEM((W + 2 * EB,), jnp.int32),
            pltpu.SemaphoreType.DMA,
            pltpu.SemaphoreType.DMA,
            pltpu.SemaphoreType.DMA,
        ],
    )(_accum_body)


def _tc_body(parts_ref, w1_ref, b1_ref, wih_ref, whh_ref, bih_ref, bhh_ref,
             ws_ref, bs_ref, out_ref, xp_ref):
    we = parts_ref[pl.ds(0, W), :]
    hw = lax.dot_general(we, w1_ref[...], (((1,), (1,)), ((), ())),
                         preferred_element_type=jnp.float32) + b1_ref[...]
    xp_ref[...] = lax.dot_general(hw, wih_ref[...], (((1,), (1,)), ((), ())),
                                  preferred_element_type=jnp.float32) \
        + bih_ref[...] + bhh_ref[...]

    def step(t, hc):
        h, cc = hc
        g = xp_ref[pl.ds(t * SP, SP), :] + lax.dot_general(
            h, whh_ref[...], (((1,), (1,)), ((), ())),
            preferred_element_type=jnp.float32)
        ii = jax.nn.sigmoid(g[:, 0:H2])
        ff = jax.nn.sigmoid(g[:, H2:2 * H2])
        gg = jnp.tanh(g[:, 2 * H2:3 * H2])
        oo = jax.nn.sigmoid(g[:, 3 * H2:4 * H2])
        cn = ff * cc + ii * gg
        hn = oo * jnp.tanh(cn)
        return (hn, cn)

    h0 = jnp.zeros((SP, H2), jnp.float32)
    h, _ = lax.fori_loop(0, L, step, (h0, h0))
    sent = jnp.maximum(h, 0.0)
    out_ref[...] = lax.dot_general(sent, ws_ref[...], (((1,), (1,)), ((), ())),
                                   preferred_element_type=jnp.float32) + bs_ref[...]


def kernel(x, edge_index, word_idx_list, W1, b1, W_ih, W_hh, b_ih, b_hh, Ws, bs):
    wp = jnp.zeros((L, SP), jnp.int32).at[:, :S].set(
        word_idx_list.astype(jnp.int32).T)
    word_flat = wp.reshape(-1)
    pos = jnp.full((NPOS,), -1, jnp.int32).at[word_flat].set(
        jnp.arange(W, dtype=jnp.int32))
    src_p = jnp.concatenate(
        [edge_index[0].astype(jnp.int32), jnp.zeros((EP - E,), jnp.int32)])
    dst_p = jnp.concatenate(
        [edge_index[1].astype(jnp.int32),
         jnp.full((EP - E,), N_NODES, jnp.int32)])

    ck, cs, cnt = _make_filter()(pos, src_p, dst_p)
    parts = _make_accum()(pos, word_flat, x, ck, cs, cnt)

    scores8 = pl.pallas_call(
        _tc_body,
        out_shape=jax.ShapeDtypeStruct((SP, 2), jnp.float32),
        scratch_shapes=[pltpu.VMEM((W, 4 * H2), jnp.float32)],
    )(parts, W1, b1.reshape(1, -1), W_ih, W_hh, b_ih.reshape(1, -1),
      b_hh.reshape(1, -1), Ws, bs.reshape(1, -1))
    return scores8[:S]


# speedup vs baseline: 1.2062x; 1.0249x over previous
import functools

import jax
import jax.numpy as jnp
from jax import lax
from jax.experimental import pallas as pl
from jax.experimental.pallas import tpu as pltpu
from jax.experimental.pallas import tpu_sc as plsc

N_NODES = 10000
D = 256
H2 = 256
E = 160000
S = 4
L = 128
SP = 8
W = L * SP
NPOS = N_NODES + 16
EP = 160256
E_PER = EP // 32
CH = E_PER // 16
BLK = 1024
CLEN = 5 * BLK
PCH = CLEN // 16
NT = 32
T_OWN = 33
DUMP = 1024
GR = 64
PG = 16
EB = 64
FLUSH_AT = 4096
PD_CAP = 5248
PACK = 16384
OUT_ROWS = W + 16


def _filter_body(pos_hbm, src_hbm, dst_hbm, ck_hbm, cnt_hbm,
                 pos_v, src_v, dst_v, ck_v, cnt_v):
    c = lax.axis_index("c")
    s = lax.axis_index("s")
    wid = s * 2 + c

    pltpu.sync_copy(pos_hbm, pos_v)
    pltpu.sync_copy(src_hbm.at[pl.ds(wid * E_PER, E_PER)], src_v)
    pltpu.sync_copy(dst_hbm.at[pl.ds(wid * E_PER, E_PER)], dst_v)

    def pf(i, carry):
        ck_v[pl.ds(i * 16, 16)] = jnp.full((16,), DUMP * PACK, jnp.int32)
        return carry
    lax.fori_loop(0, PCH, pf, 0)

    def fa(i, cur):
        dv = dst_v[pl.ds(i * 16, 16)]
        kv = plsc.load_gather(pos_v, [dv])
        m = kv >= 0
        sv = src_v[pl.ds(i * 16, 16)]
        pv = kv * PACK + sv
        plsc.store_compressed(ck_v.at[pl.ds(cur, 16)], pv, mask=m)
        return cur + jnp.sum(jnp.where(m, 1, 0).astype(jnp.int32))
    count = lax.fori_loop(0, CH, fa, jnp.int32(0))

    cnt_v[...] = jnp.full((16,), count, jnp.int32)
    pltpu.sync_copy(ck_v, ck_hbm.at[wid])
    pltpu.sync_copy(cnt_v, cnt_hbm.at[pl.ds(wid * 16, 16)])


def _accum_body(pos_hbm, word_hbm, x_hbm, ck_hbm, cnt_hbm, parts_hbm,
                pos_v, word_v, cnt_v, kst_v, kxt_v, pk_v,
                idx_a, idx_b, rows_a, rows_b, acc_v, stage_v, jidx_v,
                wj_v, wk_v, sem, sem_a, sem_b):
    c = lax.axis_index("c")
    s = lax.axis_index("s")
    tid = s * 2 + c
    base = tid * T_OWN
    lane = lax.iota(jnp.int32, 16)

    pltpu.sync_copy(pos_hbm, pos_v)
    pltpu.sync_copy(word_hbm, word_v)
    pltpu.sync_copy(cnt_hbm, cnt_v)

    def zb(i, carry):
        acc_v[pl.ds(i * 16, 16)] = jnp.zeros((16,), jnp.float32)
        return carry
    lax.fori_loop(0, (T_OWN + 1) * 16, zb, 0)

    def _mk_row_add(rows_ref):
        def row_add(r, g):
            kv = pk_v[pl.ds(g * GR + r, 16)]
            local = kv[0] // PACK - base
            for j in range(D // 16):
                chunk = rows_ref[r, pl.ds(j * 16, 16)]
                plsc.addupdate(acc_v.at[pl.ds(local * D + j * 16, 16)], chunk)
            return g
        return row_add
    row_add_a = _mk_row_add(rows_a)
    row_add_b = _mk_row_add(rows_b)

    def _fire(idx_ref, rows_ref, sm):
        def fire(g):
            for hh in range(GR // 16):
                pv = pk_v[pl.ds(g * GR + hh * 16, 16)]
                idx_ref[pl.ds(hh * 16, 16)] = pv % PACK
            pltpu.async_copy(x_hbm.at[idx_ref], rows_ref, sm)
            return jnp.int32(0)
        return fire
    fire_a = _fire(idx_a, rows_a, sem_a)
    fire_b = _fire(idx_b, rows_b, sem_b)

    def _drain(rows_ref, sm):
        def drain():
            pltpu.make_async_copy(
                x_hbm.at[pl.ds(0, GR)], rows_ref, sm).wait()
        return drain
    drain_a = _drain(rows_a, sem_a)
    drain_b = _drain(rows_b, sem_b)

    def flush_pending(pcur):
        seal = jnp.full((16,), (base + T_OWN) * PACK, jnp.int32)

        def sealw(j, carry):
            pk_v[pl.ds(pcur + j * 16, 16)] = seal
            return carry
        lax.fori_loop(0, GR // 16, sealw, 0)
        ng = (pcur + (GR - 1)) // GR

        lax.cond(ng > 0, fire_a, lambda g: jnp.int32(0), jnp.int32(0))

        def pair(u, carry):
            g1 = 2 * u + 1
            g2 = 2 * u + 2
            lax.cond(g1 < ng, fire_b, lambda g: jnp.int32(0), g1)
            drain_a()
            lax.fori_loop(0, GR, row_add_a, 2 * u)

            def do_b(_):
                lax.cond(g2 < ng, fire_a, lambda g: jnp.int32(0), g2)
                drain_b()
                lax.fori_loop(0, GR, row_add_b, g1)
                return jnp.int32(0)
            lax.cond(g1 < ng, do_b, lambda _: jnp.int32(0), 0)
            return carry
        lax.fori_loop(0, (ng + 1) // 2, pair, 0)
        return jnp.int32(0)

    def per_group(gi, pcur0):
        pltpu.sync_copy(ck_hbm.at[pl.ds(gi * PG, PG), pl.ds(0, BLK)], kst_v)
        lo = jnp.int32(base * PACK)
        hi = jnp.int32((base + T_OWN) * PACK)

        def per_p(pi, pcur1):
            p = gi * PG + pi
            cp = jnp.max(cnt_v[pl.ds(p * 16, 16)])
            e0 = jnp.minimum(jnp.int32(BLK), cp)
            nch0 = (e0 + 15) // 16

            def filt(i, cur):
                kv = kst_v[pi, pl.ds(i * 16, 16)]
                m = (kv >= lo) & (kv < hi)
                plsc.store_compressed(pk_v.at[pl.ds(cur, 16)], kv, mask=m)
                return cur + jnp.sum(jnp.where(m, 1, 0).astype(jnp.int32))
            pcur2 = lax.fori_loop(0, nch0, filt, pcur1)
            pcur2 = lax.cond(pcur2 >= FLUSH_AT, flush_pending,
                             lambda cc: cc, pcur2)

            nblk = (cp + (BLK - 1)) // BLK

            def extra(b, pcur3):
                pltpu.sync_copy(ck_hbm.at[p, pl.ds(b * BLK, BLK)], kxt_v)
                eb = jnp.minimum(jnp.int32(BLK), cp - b * BLK)
                nch = (eb + 15) // 16

                def filtx(i, cur):
                    kv = kxt_v[pl.ds(i * 16, 16)]
                    m = (kv >= lo) & (kv < hi)
                    plsc.store_compressed(pk_v.at[pl.ds(cur, 16)], kv, mask=m)
                    return cur + jnp.sum(jnp.where(m, 1, 0).astype(jnp.int32))
                pcur4 = lax.fori_loop(0, nch, filtx, pcur3)
                return lax.cond(pcur4 >= FLUSH_AT, flush_pending,
                                lambda cc: cc, pcur4)
            return lax.fori_loop(1, nblk, extra, pcur2)
        return lax.fori_loop(0, PG, per_p, pcur0)
    pend = lax.fori_loop(0, NT // PG, per_group, jnp.int32(0))
    flush_pending(pend)

    def wscan(q, cur):
        wv = word_v[pl.ds(q * 16, 16)]
        kj = plsc.load_gather(pos_v, [wv])
        m = (kj >= base) & (kj < base + T_OWN)
        jv = lane + q * 16
        plsc.store_compressed(wj_v.at[pl.ds(cur, 16)], jv, mask=m)
        plsc.store_compressed(wk_v.at[pl.ds(cur, 16)], kj, mask=m)
        return cur + jnp.sum(jnp.where(m, 1, 0).astype(jnp.int32))
    ccur = lax.fori_loop(0, W // 16, wscan, jnp.int32(0))

    def sealw2(j, carry):
        wj_v[pl.ds(ccur + j * 16, 16)] = jnp.full((16,), W, jnp.int32) + lane
        wk_v[pl.ds(ccur + j * 16, 16)] = jnp.full((16,), base, jnp.int32)
        return carry
    lax.fori_loop(0, EB // 16, sealw2, 0)

    def stage_row(r, g):
        kv = wk_v[pl.ds(g * EB + r, 16)]
        kr = kv[0]
        local = kr - base
        for j in range(D // 16):
            stage_v[r, pl.ds(j * 16, 16)] = \
                acc_v[pl.ds(local * D + j * 16, 16)]
        return g

    def emit_chunk(g, carry):
        lax.fori_loop(0, EB, stage_row, g)
        for hh in range(EB // 16):
            jidx_v[pl.ds(hh * 16, 16)] = wj_v[pl.ds(g * EB + hh * 16, 16)]
        pltpu.async_copy(stage_v, parts_hbm.at[jidx_v], sem).wait()
        return carry
    ne = (ccur + (EB - 1)) // EB
    lax.fori_loop(0, ne, emit_chunk, 0)


def _make_filter():
    return functools.partial(
        pl.kernel,
        mesh=plsc.VectorSubcoreMesh(core_axis_name="c", subcore_axis_name="s"),
        out_type=(
            jax.ShapeDtypeStruct((NT, CLEN), jnp.int32),
            jax.ShapeDtypeStruct((NT * 16,), jnp.int32),
        ),
        compiler_params=pltpu.CompilerParams(needs_layout_passes=False),
        scratch_types=[
            pltpu.VMEM((NPOS,), jnp.int32),
            pltpu.VMEM((E_PER,), jnp.int32),
            pltpu.VMEM((E_PER,), jnp.int32),
            pltpu.VMEM((CLEN,), jnp.int32),
            pltpu.VMEM((16,), jnp.int32),
        ],
    )(_filter_body)


def _make_accum():
    return functools.partial(
        pl.kernel,
        mesh=plsc.VectorSubcoreMesh(core_axis_name="c", subcore_axis_name="s"),
        out_type=jax.ShapeDtypeStruct((OUT_ROWS, D), jnp.float32),
        compiler_params=pltpu.CompilerParams(needs_layout_passes=False),
        scratch_types=[
            pltpu.VMEM((NPOS,), jnp.int32),
            pltpu.VMEM((W,), jnp.int32),
            pltpu.VMEM((NT * 16,), jnp.int32),
            pltpu.VMEM((PG, BLK), jnp.int32),
            pltpu.VMEM((BLK,), jnp.int32),
            pltpu.VMEM((PD_CAP,), jnp.int32),
            pltpu.VMEM((GR,), jnp.int32),
            pltpu.VMEM((GR,), jnp.int32),
            pltpu.VMEM((GR, D), jnp.float32),
            pltpu.VMEM((GR, D), jnp.float32),
            pltpu.VMEM(((T_OWN + 1) * D,), jnp.float32),
            pltpu.VMEM((EB, D), jnp.float32),
            pltpu.VMEM((EB,), jnp.int32),
            pltpu.VMEM((W + 2 * EB,), jnp.int32),
            pltpu.VMEM((W + 2 * EB,), jnp.int32),
            pltpu.SemaphoreType.DMA,
            pltpu.SemaphoreType.DMA,
            pltpu.SemaphoreType.DMA,
        ],
    )(_accum_body)


def _tc_body(parts_ref, w1_ref, b1_ref, wih_ref, whh_ref, bih_ref, bhh_ref,
             ws_ref, bs_ref, out_ref, xp_ref):
    we = parts_ref[pl.ds(0, W), :]
    hw = lax.dot_general(we, w1_ref[...], (((1,), (1,)), ((), ())),
                         preferred_element_type=jnp.float32) + b1_ref[...]
    xp_ref[...] = lax.dot_general(hw, wih_ref[...], (((1,), (1,)), ((), ())),
                                  preferred_element_type=jnp.float32) \
        + bih_ref[...] + bhh_ref[...]

    def step(t, hc):
        h, cc = hc
        g = xp_ref[pl.ds(t * SP, SP), :] + lax.dot_general(
            h, whh_ref[...], (((1,), (1,)), ((), ())),
            preferred_element_type=jnp.float32)
        ii = jax.nn.sigmoid(g[:, 0:H2])
        ff = jax.nn.sigmoid(g[:, H2:2 * H2])
        gg = jnp.tanh(g[:, 2 * H2:3 * H2])
        oo = jax.nn.sigmoid(g[:, 3 * H2:4 * H2])
        cn = ff * cc + ii * gg
        hn = oo * jnp.tanh(cn)
        return (hn, cn)

    h0 = jnp.zeros((SP, H2), jnp.float32)
    h, _ = lax.fori_loop(0, L, step, (h0, h0))
    sent = jnp.maximum(h, 0.0)
    out_ref[...] = lax.dot_general(sent, ws_ref[...], (((1,), (1,)), ((), ())),
                                   preferred_element_type=jnp.float32) + bs_ref[...]


def kernel(x, edge_index, word_idx_list, W1, b1, W_ih, W_hh, b_ih, b_hh, Ws, bs):
    wp = jnp.zeros((L, SP), jnp.int32).at[:, :S].set(
        word_idx_list.astype(jnp.int32).T)
    word_flat = wp.reshape(-1)
    pos = jnp.full((NPOS,), -1, jnp.int32).at[word_flat].set(
        jnp.arange(W, dtype=jnp.int32))
    src_p = jnp.concatenate(
        [edge_index[0].astype(jnp.int32), jnp.zeros((EP - E,), jnp.int32)])
    dst_p = jnp.concatenate(
        [edge_index[1].astype(jnp.int32),
         jnp.full((EP - E,), N_NODES, jnp.int32)])

    ck, cnt = _make_filter()(pos, src_p, dst_p)
    parts = _make_accum()(pos, word_flat, x, ck, cnt)

    scores8 = pl.pallas_call(
        _tc_body,
        out_shape=jax.ShapeDtypeStruct((SP, 2), jnp.float32),
        scratch_shapes=[pltpu.VMEM((W, 4 * H2), jnp.float32)],
    )(parts, W1, b1.reshape(1, -1), W_ih, W_hh, b_ih.reshape(1, -1),
      b_hh.reshape(1, -1), Ws, bs.reshape(1, -1))
    return scores8[:S]
